# Initial kernel scaffold; baseline (speedup 1.0000x reference)
import jax
import jax.numpy as jnp
from jax.experimental import pallas as pl


def kernel(x, edge_index, W_l0, b_l0, W_r0, gamma0, beta0, W_l1, b_l1, W_r1, gamma1, beta1, W_fc, b_fc):
    raise NotImplementedError("write your pallas kernel here")


# speedup vs baseline: 4.3291x; 4.3291x over previous
import functools

import jax
import jax.numpy as jnp
from jax import lax
from jax.experimental import pallas as pl
from jax.experimental.pallas import tpu as pltpu
from jax.experimental.pallas import tpu_sc as plsc

N = 10000
E = 320000
D = 128
C = 47

NC = 2
NS = 16
NW = NC * NS
CH = 128
J = -(-E // (NW * CH))
EP = NW * J * CH
EPW = J * CH
NP = 10240
SLAB = NP // NS


@functools.lru_cache(maxsize=None)
def _make_sc_agg(with_counts: bool):
    out_type = [jax.ShapeDtypeStruct((NC, NP, D), jnp.float32)]
    if with_counts:
        out_type.append(jax.ShapeDtypeStruct((NC, NP), jnp.float32))
    mesh = plsc.VectorSubcoreMesh(core_axis_name="c", subcore_axis_name="s")

    def body(table, src3, dst3, z2d, z1d, ones1d, *refs):
        if with_counts:
            (psums, pcnt, src_v, dst_v, rows_v, ones_v, acc_sh, cnt_sh,
             sem) = refs
        else:
            (psums, src_v, dst_v, rows_v, acc_sh, sem) = refs
        c = lax.axis_index("c")
        s = lax.axis_index("s")
        wid = s * NC + c

        pltpu.sync_copy(src3.at[wid], src_v)
        pltpu.sync_copy(dst3.at[wid], dst_v)
        if with_counts:
            pltpu.sync_copy(ones1d, ones_v)

        r0 = s * SLAB
        pltpu.sync_copy(z2d, acc_sh.at[pl.ds(r0, SLAB)])
        if with_counts:
            pltpu.sync_copy(z1d, cnt_sh.at[pl.ds(r0, SLAB)])
        plsc.subcore_barrier()

        def step(j, carry):
            srow = src_v.at[j]
            drow = dst_v.at[j]
            pltpu.async_copy(table.at[srow], rows_v, sem).wait()
            pltpu.sync_copy(rows_v, acc_sh.at[drow], add=True)
            if with_counts:
                pltpu.sync_copy(ones_v, cnt_sh.at[drow], add=True)
            return carry

        lax.fori_loop(0, J, step, 0)

        plsc.subcore_barrier()
        pltpu.sync_copy(acc_sh.at[pl.ds(r0, SLAB)],
                        psums.at[c].at[pl.ds(r0, SLAB)])
        if with_counts:
            pltpu.sync_copy(cnt_sh.at[pl.ds(r0, SLAB)],
                            pcnt.at[c].at[pl.ds(r0, SLAB)])

    scratch = [
        pltpu.VMEM((J, CH), jnp.int32),
        pltpu.VMEM((J, CH), jnp.int32),
        pltpu.VMEM((CH, D), jnp.float32),
    ]
    if with_counts:
        scratch.append(pltpu.VMEM((CH,), jnp.float32))
    scratch.append(pltpu.VMEM_SHARED((NP, D), jnp.float32))
    if with_counts:
        scratch.append(pltpu.VMEM_SHARED((NP,), jnp.float32))
    scratch.append(pltpu.SemaphoreType.DMA)

    return pl.kernel(body, out_type=tuple(out_type), mesh=mesh,
                     scratch_types=tuple(scratch))


def _dense_body(with_fc, *refs):
    if with_fc:
        (ps, pc, x, wl, bl, wr, g, b, wfc, bfc, o) = refs
    else:
        (ps, pc, x, wl, bl, wr, g, b, o) = refs
    sums = ps[0][:N, :] + ps[1][:N, :]
    cnt = pc[0][:N, :] + pc[1][:N, :]
    agg = sums * (1.0 / jnp.maximum(cnt, 1.0))
    h = jax.lax.dot(agg, wl[:], preferred_element_type=jnp.float32)
    h = h + bl[:]
    h = h + jax.lax.dot(x[:], wr[:], preferred_element_type=jnp.float32)
    norm = jnp.sqrt(jnp.sum(h * h, axis=1, keepdims=True))
    h = h / jnp.maximum(norm, 1e-12)
    mu = jnp.mean(h, axis=0, keepdims=True)
    var = jnp.mean((h - mu) * (h - mu), axis=0, keepdims=True)
    h = g[:] * (h - mu) / jnp.sqrt(var + 1e-5) + b[:]
    h = jnp.maximum(h, 0.0)
    if with_fc:
        h = jax.lax.dot(h, wfc[:], preferred_element_type=jnp.float32)
        h = h + bfc[:]
    o[:] = h


_dense = pl.pallas_call(
    functools.partial(_dense_body, False),
    out_shape=jax.ShapeDtypeStruct((N, D), jnp.float32),
)

_dense_fc = pl.pallas_call(
    functools.partial(_dense_body, True),
    out_shape=jax.ShapeDtypeStruct((N, D), jnp.float32),
)


def kernel(x, edge_index, W_l0, b_l0, W_r0, gamma0, beta0,
           W_l1, b_l1, W_r1, gamma1, beta1, W_fc, b_fc):
    dst = edge_index[0].astype(jnp.int32)
    src = edge_index[1].astype(jnp.int32)
    src3 = jnp.concatenate(
        [src, jnp.zeros((EP - E,), jnp.int32)]).reshape(NW, J, CH)
    dst3 = jnp.concatenate(
        [dst, jnp.full((EP - E,), N, jnp.int32)]).reshape(NW, J, CH)
    z2d = jnp.zeros((SLAB, D), jnp.float32)
    z1d = jnp.zeros((SLAB,), jnp.float32)
    ones1d = jnp.ones((CH,), jnp.float32)

    ps0, pc = _make_sc_agg(True)(x, src3, dst3, z2d, z1d, ones1d)
    pc_col = pc[:, :, None]
    h0 = _dense(ps0, pc_col, x, W_l0, b_l0.reshape(1, D), W_r0,
                gamma0.reshape(1, D), beta0.reshape(1, D))
    (ps1,) = _make_sc_agg(False)(h0, src3, dst3, z2d, z1d, ones1d)
    wfc_p = jnp.pad(W_fc, ((0, 0), (0, D - C)))
    bfc_p = jnp.pad(b_fc, (0, D - C)).reshape(1, D)
    out_p = _dense_fc(ps1, pc_col, h0, W_l1, b_l1.reshape(1, D), W_r1,
                      gamma1.reshape(1, D), beta1.reshape(1, D),
                      wfc_p, bfc_p)
    return out_p[:, :C]


# speedup vs baseline: 4.7575x; 1.0990x over previous
import functools

import jax
import jax.numpy as jnp
from jax import lax
from jax.experimental import pallas as pl
from jax.experimental.pallas import tpu as pltpu
from jax.experimental.pallas import tpu_sc as plsc

N = 10000
E = 320000
D = 128
C = 47

NC = 2
NS = 16
NW = NC * NS
CH = 128
J = 79
EP = NW * J * CH
NP = 10240
SLAB = NP // NS
BINS = NP // 2

assert J * CH * NW >= E and J % 2 == 1

_SC_PARAMS = None


def _sc_params():
    return pltpu.CompilerParams(needs_layout_passes=False)


@functools.lru_cache(maxsize=None)
def _make_sc_agg():
    mesh = plsc.VectorSubcoreMesh(core_axis_name="c", subcore_axis_name="s")

    def body(table, src1d, dst3, z2d, psums, dst_v, srcdb, rows_v,
             sem0, sem1, semi0, semi1, acc_sh):
        c = lax.axis_index("c")
        s = lax.axis_index("s")
        wid = s * NC + c

        pltpu.sync_copy(dst3.at[wid], dst_v)

        r0 = s * SLAB
        pltpu.sync_copy(z2d, acc_sh.at[pl.ds(r0, SLAB)])
        plsc.subcore_barrier()

        rows0 = rows_v.at[0]
        rows1 = rows_v.at[1]
        src0 = srcdb.at[pl.ds(0, CH)]
        src1 = srcdb.at[pl.ds(CH, CH)]
        base_e = wid * (J * CH)

        def idx_fetch(j, buf, sem):
            jc = jnp.minimum(j, J - 1)
            return pltpu.make_async_copy(
                src1d.at[pl.ds(base_e + jc * CH, CH)], buf, sem)

        def gather(buf_idx, buf, sem):
            return pltpu.make_async_copy(table.at[buf_idx], buf, sem)

        idx_fetch(0, src0, semi0).start()
        idx_fetch(0, src0, semi0).wait()
        gather(src0, rows0, sem0).start()
        idx_fetch(1, src1, semi1).start()

        def step2(i, carry):
            j0 = 2 * i
            idx_fetch(j0 + 1, src1, semi1).wait()
            gather(src1, rows1, sem1).start()
            gather(src0, rows0, sem0).wait()
            idx_fetch(j0 + 2, src0, semi0).start()
            pltpu.sync_copy(rows0, acc_sh.at[dst_v.at[j0]], add=True)
            idx_fetch(j0 + 2, src0, semi0).wait()
            gather(src0, rows0, sem0).start()
            gather(src1, rows1, sem1).wait()
            idx_fetch(j0 + 3, src1, semi1).start()
            pltpu.sync_copy(rows1, acc_sh.at[dst_v.at[j0 + 1]], add=True)
            return carry

        lax.fori_loop(0, (J - 1) // 2, step2, 0)
        gather(src0, rows0, sem0).wait()
        idx_fetch(J - 1, src1, semi1).wait()
        pltpu.sync_copy(rows0, acc_sh.at[dst_v.at[J - 1]], add=True)

        plsc.subcore_barrier()
        pltpu.sync_copy(acc_sh.at[pl.ds(r0, SLAB)],
                        psums.at[c].at[pl.ds(r0, SLAB)])

    return pl.kernel(
        body,
        out_type=(jax.ShapeDtypeStruct((NC, NP, D), jnp.float32),),
        mesh=mesh,
        scratch_types=(
            pltpu.VMEM((J, CH), jnp.int32),
            pltpu.VMEM((2 * CH,), jnp.int32),
            pltpu.VMEM((2, CH, D), jnp.float32),
            pltpu.SemaphoreType.DMA,
            pltpu.SemaphoreType.DMA,
            pltpu.SemaphoreType.DMA,
            pltpu.SemaphoreType.DMA,
            pltpu.VMEM_SHARED((NP, D), jnp.float32),
        ),
        compiler_params=_sc_params())


@functools.lru_cache(maxsize=None)
def _make_sc_cnt():
    mesh = plsc.VectorSubcoreMesh(core_axis_name="c", subcore_axis_name="s")

    def body(dst3, pcnt, dst_v, hist, cntbuf, slab_v, sem, hist_sh):
        c = lax.axis_index("c")
        s = lax.axis_index("s")
        wid = s * NC + c

        pltpu.sync_copy(dst3.at[wid], dst_v)
        lane = lax.iota(jnp.int32, 16)
        ones16 = jnp.ones((16,), jnp.float32)
        zeros16 = jnp.zeros((16,), jnp.float32)

        for p in range(NP // BINS):
            base = p * BINS

            def zero_row(i, carry):
                hist[carry, pl.ds(i * 16, 16)] = zeros16
                return carry

            for l in range(16):
                lax.fori_loop(0, BINS // 16, zero_row, l)

            def feed(j, carry):
                for k in range(CH // 16):
                    idx = dst_v[j, pl.ds(k * 16, 16)]
                    rel = idx - base
                    m = jnp.logical_and(rel >= 0, rel < BINS)
                    relc = jnp.minimum(jnp.maximum(rel, 0), BINS - 1)
                    plsc.addupdate_scatter(hist, [lane, relc], ones16,
                                           mask=m)
                return carry

            lax.fori_loop(0, J, feed, 0)

            def reduce_cols(ci, carry):
                sl = pl.ds(ci * 16, 16)
                v = hist[0, sl]
                for l in range(1, 16):
                    v = v + hist[l, sl]
                cntbuf[sl] = v
                return carry

            lax.fori_loop(0, BINS // 16, reduce_cols, 0)
            pltpu.sync_copy(cntbuf, hist_sh.at[s].at[0].at[pl.ds(base, BINS)])

        plsc.subcore_barrier()
        r0 = s * SLAB
        pltpu.sync_copy(hist_sh.at[:, 0, pl.ds(r0, SLAB)], slab_v)

        def reduce_slab(ci, carry):
            sl = pl.ds(ci * 16, 16)
            v = slab_v[0, sl]
            for l in range(1, 16):
                v = v + slab_v[l, sl]
            cntbuf[sl] = v
            return carry

        lax.fori_loop(0, SLAB // 16, reduce_slab, 0)
        pltpu.sync_copy(cntbuf.at[pl.ds(0, SLAB)],
                        pcnt.at[c].at[0].at[pl.ds(r0, SLAB)])

    return pl.kernel(
        body,
        out_type=(jax.ShapeDtypeStruct((NC, 1, NP), jnp.float32),),
        mesh=mesh,
        scratch_types=(
            pltpu.VMEM((J, CH), jnp.int32),
            pltpu.VMEM((16, BINS), jnp.float32),
            pltpu.VMEM((BINS,), jnp.float32),
            pltpu.VMEM((16, SLAB), jnp.float32),
            pltpu.SemaphoreType.DMA,
            pltpu.VMEM_SHARED((16, 1, NP), jnp.float32),
        ),
        compiler_params=_sc_params())


def _dense0_body(ps, pc, x, wl, bl, wr, g, b, o, ocnt):
    sums = ps[0, :N, :] + ps[1, :N, :]
    cnt = pc[0, :N, :] + pc[1, :N, :]
    ocnt[:] = cnt
    agg = sums * (1.0 / jnp.maximum(cnt, 1.0))
    h = jax.lax.dot(agg, wl[:], preferred_element_type=jnp.float32)
    h = h + bl[:]
    h = h + jax.lax.dot(x[:], wr[:], preferred_element_type=jnp.float32)
    norm = jnp.sqrt(jnp.sum(h * h, axis=1, keepdims=True))
    h = h / jnp.maximum(norm, 1e-12)
    mu = jnp.mean(h, axis=0, keepdims=True)
    var = jnp.mean((h - mu) * (h - mu), axis=0, keepdims=True)
    h = g[:] * (h - mu) / jnp.sqrt(var + 1e-5) + b[:]
    o[:] = jnp.maximum(h, 0.0)


def _dense1_body(ps, cnt_ref, x, wl, bl, wr, g, b, wfc, bfc, o):
    sums = ps[0, :N, :] + ps[1, :N, :]
    cnt = cnt_ref[:]
    agg = sums * (1.0 / jnp.maximum(cnt, 1.0))
    h = jax.lax.dot(agg, wl[:], preferred_element_type=jnp.float32)
    h = h + bl[:]
    h = h + jax.lax.dot(x[:], wr[:], preferred_element_type=jnp.float32)
    norm = jnp.sqrt(jnp.sum(h * h, axis=1, keepdims=True))
    h = h / jnp.maximum(norm, 1e-12)
    mu = jnp.mean(h, axis=0, keepdims=True)
    var = jnp.mean((h - mu) * (h - mu), axis=0, keepdims=True)
    h = g[:] * (h - mu) / jnp.sqrt(var + 1e-5) + b[:]
    h = jnp.maximum(h, 0.0)
    h = jax.lax.dot(h, wfc[:], preferred_element_type=jnp.float32)
    o[:] = h + bfc[:]


_dense0 = pl.pallas_call(
    _dense0_body,
    out_shape=(jax.ShapeDtypeStruct((N, D), jnp.float32),
               jax.ShapeDtypeStruct((N, 1), jnp.float32)),
)

_dense1 = pl.pallas_call(
    _dense1_body,
    out_shape=jax.ShapeDtypeStruct((N, D), jnp.float32),
)


def kernel(x, edge_index, W_l0, b_l0, W_r0, gamma0, beta0,
           W_l1, b_l1, W_r1, gamma1, beta1, W_fc, b_fc):
    dst = edge_index[0].astype(jnp.int32)
    src = edge_index[1].astype(jnp.int32)
    src1d = jnp.concatenate([src, jnp.zeros((EP - E,), jnp.int32)])
    dst3 = jnp.concatenate(
        [dst, jnp.full((EP - E,), N, jnp.int32)]).reshape(NW, J, CH)
    z2d = jnp.zeros((SLAB, D), jnp.float32)

    (ps0,) = _make_sc_agg()(x, src1d, dst3, z2d)
    (pc3,) = _make_sc_cnt()(dst3)
    h0, cnt_col = _dense0(ps0, pc3[:, 0, :, None], x, W_l0, b_l0.reshape(1, D),
                          W_r0, gamma0.reshape(1, D), beta0.reshape(1, D))
    (ps1,) = _make_sc_agg()(h0, src1d, dst3, z2d)
    wfc_p = jnp.pad(W_fc, ((0, 0), (0, D - C)))
    bfc_p = jnp.pad(b_fc, (0, D - C)).reshape(1, D)
    out_p = _dense1(ps1, cnt_col, h0, W_l1, b_l1.reshape(1, D), W_r1,
                    gamma1.reshape(1, D), beta1.reshape(1, D),
                    wfc_p, bfc_p)
    return out_p[:, :C]


# speedup vs baseline: 5.7099x; 1.2002x over previous
import functools

import jax
import jax.numpy as jnp
from jax import lax
from jax.experimental import pallas as pl
from jax.experimental.pallas import tpu as pltpu
from jax.experimental.pallas import tpu_sc as plsc

N = 10000
E = 320000
D = 128
C = 47

NC = 2
NS = 16
NW = NC * NS
CH = 128
J = 79
EP = NW * J * CH
NP = 10240
SLAB = NP // NS
BINS = NP // 2

J0 = 113
J1 = 45

assert J * CH * NW >= E and J % 2 == 1
assert (J0 + J1) * NS * CH == EP and J0 % 2 == 1 and J1 % 2 == 1
assert J0 * NS * CH <= E

_SC_PARAMS = None


def _sc_params():
    return pltpu.CompilerParams(needs_layout_passes=False)


@functools.lru_cache(maxsize=None)
def _make_sc_agg():
    mesh = plsc.VectorSubcoreMesh(core_axis_name="c", subcore_axis_name="s")

    def body(table, src1d, dst_a, dst_b, z2d, psums, dst_v, srcdb, rows_v,
             sem0, sem1, semi0, semi1, acc_sh):
        c = lax.axis_index("c")
        s = lax.axis_index("s")

        r0 = s * SLAB
        pltpu.sync_copy(z2d, acc_sh.at[pl.ds(r0, SLAB)])

        rows0 = rows_v.at[0]
        rows1 = rows_v.at[1]
        src0 = srcdb.at[pl.ds(0, CH)]
        src1 = srcdb.at[pl.ds(CH, CH)]

        def run_stream(jn, base_e):
            def idx_fetch(j, buf, sem):
                jc = jnp.minimum(j, jn - 1)
                return pltpu.make_async_copy(
                    src1d.at[pl.ds(base_e + jc * CH, CH)], buf, sem)

            def gather(buf_idx, buf, sem):
                return pltpu.make_async_copy(table.at[buf_idx], buf, sem)

            idx_fetch(0, src0, semi0).start()
            idx_fetch(0, src0, semi0).wait()
            gather(src0, rows0, sem0).start()
            idx_fetch(1, src1, semi1).start()

            def step2(i, carry):
                j0 = 2 * i
                idx_fetch(j0 + 1, src1, semi1).wait()
                gather(src1, rows1, sem1).start()
                gather(src0, rows0, sem0).wait()
                idx_fetch(j0 + 2, src0, semi0).start()
                pltpu.sync_copy(rows0, acc_sh.at[dst_v.at[j0]], add=True)
                idx_fetch(j0 + 2, src0, semi0).wait()
                gather(src0, rows0, sem0).start()
                gather(src1, rows1, sem1).wait()
                idx_fetch(j0 + 3, src1, semi1).start()
                pltpu.sync_copy(rows1, acc_sh.at[dst_v.at[j0 + 1]],
                                add=True)
                return carry

            lax.fori_loop(0, (jn - 1) // 2, step2, 0)
            gather(src0, rows0, sem0).wait()
            idx_fetch(jn - 1, src1, semi1).wait()
            pltpu.sync_copy(rows0, acc_sh.at[dst_v.at[jn - 1]], add=True)

        @pl.when(c == 0)
        def _():
            pltpu.sync_copy(dst_a.at[s], dst_v.at[pl.ds(0, J0)])

        @pl.when(c == 1)
        def _():
            pltpu.sync_copy(dst_b.at[s], dst_v.at[pl.ds(0, J1)])

        plsc.subcore_barrier()

        @pl.when(c == 0)
        def _():
            run_stream(J0, s * (J0 * CH))

        @pl.when(c == 1)
        def _():
            run_stream(J1, NS * J0 * CH + s * (J1 * CH))

        plsc.subcore_barrier()
        pltpu.sync_copy(acc_sh.at[pl.ds(r0, SLAB)],
                        psums.at[c].at[pl.ds(r0, SLAB)])

    return pl.kernel(
        body,
        out_type=(jax.ShapeDtypeStruct((NC, NP, D), jnp.float32),),
        mesh=mesh,
        scratch_types=(
            pltpu.VMEM((J0, CH), jnp.int32),
            pltpu.VMEM((2 * CH,), jnp.int32),
            pltpu.VMEM((2, CH, D), jnp.float32),
            pltpu.SemaphoreType.DMA,
            pltpu.SemaphoreType.DMA,
            pltpu.SemaphoreType.DMA,
            pltpu.SemaphoreType.DMA,
            pltpu.VMEM_SHARED((NP, D), jnp.float32),
        ),
        compiler_params=_sc_params())


@functools.lru_cache(maxsize=None)
def _make_sc_cnt():
    mesh = plsc.VectorSubcoreMesh(core_axis_name="c", subcore_axis_name="s")

    def body(dst3, pcnt, dst_v, hist, cntbuf, slab_v, sem, hist_sh):
        c = lax.axis_index("c")
        s = lax.axis_index("s")
        wid = s * NC + c

        pltpu.sync_copy(dst3.at[wid], dst_v)
        lane = lax.iota(jnp.int32, 16)
        ones16 = jnp.ones((16,), jnp.float32)
        zeros16 = jnp.zeros((16,), jnp.float32)

        for p in range(NP // BINS):
            base = p * BINS

            def zero_row(i, carry):
                hist[carry, pl.ds(i * 16, 16)] = zeros16
                return carry

            for l in range(16):
                lax.fori_loop(0, BINS // 16, zero_row, l)

            def feed(j, carry):
                for k in range(CH // 16):
                    idx = dst_v[j, pl.ds(k * 16, 16)]
                    rel = idx - base
                    m = jnp.logical_and(rel >= 0, rel < BINS)
                    relc = jnp.minimum(jnp.maximum(rel, 0), BINS - 1)
                    plsc.addupdate_scatter(hist, [lane, relc], ones16,
                                           mask=m)
                return carry

            lax.fori_loop(0, J, feed, 0)

            def reduce_cols(ci, carry):
                sl = pl.ds(ci * 16, 16)
                v = hist[0, sl]
                for l in range(1, 16):
                    v = v + hist[l, sl]
                cntbuf[sl] = v
                return carry

            lax.fori_loop(0, BINS // 16, reduce_cols, 0)
            pltpu.sync_copy(cntbuf, hist_sh.at[s].at[0].at[pl.ds(base, BINS)])

        plsc.subcore_barrier()
        r0 = s * SLAB
        pltpu.sync_copy(hist_sh.at[:, 0, pl.ds(r0, SLAB)], slab_v)

        def reduce_slab(ci, carry):
            sl = pl.ds(ci * 16, 16)
            v = slab_v[0, sl]
            for l in range(1, 16):
                v = v + slab_v[l, sl]
            cntbuf[sl] = v
            return carry

        lax.fori_loop(0, SLAB // 16, reduce_slab, 0)
        pltpu.sync_copy(cntbuf.at[pl.ds(0, SLAB)],
                        pcnt.at[c].at[0].at[pl.ds(r0, SLAB)])

    return pl.kernel(
        body,
        out_type=(jax.ShapeDtypeStruct((NC, 1, NP), jnp.float32),),
        mesh=mesh,
        scratch_types=(
            pltpu.VMEM((J, CH), jnp.int32),
            pltpu.VMEM((16, BINS), jnp.float32),
            pltpu.VMEM((BINS,), jnp.float32),
            pltpu.VMEM((16, SLAB), jnp.float32),
            pltpu.SemaphoreType.DMA,
            pltpu.VMEM_SHARED((16, 1, NP), jnp.float32),
        ),
        compiler_params=_sc_params())


def _dense0_body(ps, pc, x, wl, bl, wr, g, b, o, ocnt):
    sums = ps[0, :N, :] + ps[1, :N, :]
    cnt = pc[0, :N, :] + pc[1, :N, :]
    ocnt[:] = cnt
    agg = sums * (1.0 / jnp.maximum(cnt, 1.0))
    h = jax.lax.dot(agg, wl[:], preferred_element_type=jnp.float32)
    h = h + bl[:]
    h = h + jax.lax.dot(x[:], wr[:], preferred_element_type=jnp.float32)
    norm = jnp.sqrt(jnp.sum(h * h, axis=1, keepdims=True))
    h = h / jnp.maximum(norm, 1e-12)
    mu = jnp.mean(h, axis=0, keepdims=True)
    var = jnp.mean((h - mu) * (h - mu), axis=0, keepdims=True)
    h = g[:] * (h - mu) / jnp.sqrt(var + 1e-5) + b[:]
    o[:] = jnp.maximum(h, 0.0)


def _dense1_body(ps, cnt_ref, x, wl, bl, wr, g, b, wfc, bfc, o):
    sums = ps[0, :N, :] + ps[1, :N, :]
    cnt = cnt_ref[:]
    agg = sums * (1.0 / jnp.maximum(cnt, 1.0))
    h = jax.lax.dot(agg, wl[:], preferred_element_type=jnp.float32)
    h = h + bl[:]
    h = h + jax.lax.dot(x[:], wr[:], preferred_element_type=jnp.float32)
    norm = jnp.sqrt(jnp.sum(h * h, axis=1, keepdims=True))
    h = h / jnp.maximum(norm, 1e-12)
    mu = jnp.mean(h, axis=0, keepdims=True)
    var = jnp.mean((h - mu) * (h - mu), axis=0, keepdims=True)
    h = g[:] * (h - mu) / jnp.sqrt(var + 1e-5) + b[:]
    h = jnp.maximum(h, 0.0)
    h = jax.lax.dot(h, wfc[:], preferred_element_type=jnp.float32)
    o[:] = h + bfc[:]


_dense0 = pl.pallas_call(
    _dense0_body,
    out_shape=(jax.ShapeDtypeStruct((N, D), jnp.float32),
               jax.ShapeDtypeStruct((N, 1), jnp.float32)),
)

_dense1 = pl.pallas_call(
    _dense1_body,
    out_shape=jax.ShapeDtypeStruct((N, D), jnp.float32),
)


def kernel(x, edge_index, W_l0, b_l0, W_r0, gamma0, beta0,
           W_l1, b_l1, W_r1, gamma1, beta1, W_fc, b_fc):
    dst = edge_index[0].astype(jnp.int32)
    src = edge_index[1].astype(jnp.int32)
    src1d = jnp.concatenate([src, jnp.zeros((EP - E,), jnp.int32)])
    dstp = jnp.concatenate([dst, jnp.full((EP - E,), N, jnp.int32)])
    dst3 = dstp.reshape(NW, J, CH)
    dst_a = dstp[:NS * J0 * CH].reshape(NS, J0, CH)
    dst_b = dstp[NS * J0 * CH:].reshape(NS, J1, CH)
    z2d = jnp.zeros((SLAB, D), jnp.float32)

    (ps0,) = _make_sc_agg()(x, src1d, dst_a, dst_b, z2d)
    (pc3,) = _make_sc_cnt()(dst3)
    h0, cnt_col = _dense0(ps0, pc3[:, 0, :, None], x, W_l0, b_l0.reshape(1, D),
                          W_r0, gamma0.reshape(1, D), beta0.reshape(1, D))
    (ps1,) = _make_sc_agg()(h0, src1d, dst_a, dst_b, z2d)
    wfc_p = jnp.pad(W_fc, ((0, 0), (0, D - C)))
    bfc_p = jnp.pad(b_fc, (0, D - C)).reshape(1, D)
    out_p = _dense1(ps1, cnt_col, h0, W_l1, b_l1.reshape(1, D), W_r1,
                    gamma1.reshape(1, D), beta1.reshape(1, D),
                    wfc_p, bfc_p)
    return out_p[:, :C]


# speedup vs baseline: 5.7296x; 1.0035x over previous
import functools

import jax
import jax.numpy as jnp
from jax import lax
from jax.experimental import pallas as pl
from jax.experimental.pallas import tpu as pltpu
from jax.experimental.pallas import tpu_sc as plsc

N = 10000
E = 320000
D = 128
C = 47

NC = 2
NS = 16
NW = NC * NS
CH = 128
J = 79
EP = NW * J * CH
NP = 10240
SLAB = NP // NS
BINS = NP // 2

J0 = 113
J1 = 45

assert J * CH * NW >= E and J % 2 == 1
assert (J0 + J1) * NS * CH == EP and J0 % 2 == 1 and J1 % 2 == 1
assert J0 * NS * CH <= E

_SC_PARAMS = None


def _sc_params():
    return pltpu.CompilerParams(needs_layout_passes=False)


@functools.lru_cache(maxsize=None)
def _make_sc_agg():
    mesh = plsc.VectorSubcoreMesh(core_axis_name="c", subcore_axis_name="s")

    def body(table, src1d, dst_a, dst_b, z2d, psums, dst_v, srcdb, rows_v,
             sem0, sem1, semi0, semi1, acc_sh):
        c = lax.axis_index("c")
        s = lax.axis_index("s")

        r0 = s * SLAB
        pltpu.sync_copy(z2d, acc_sh.at[pl.ds(r0, SLAB)])

        rows0 = rows_v.at[0]
        rows1 = rows_v.at[1]
        src0 = srcdb.at[pl.ds(0, CH)]
        src1 = srcdb.at[pl.ds(CH, CH)]

        def run_stream(jn, base_e):
            def idx_fetch(j, buf, sem):
                jc = jnp.minimum(j, jn - 1)
                return pltpu.make_async_copy(
                    src1d.at[pl.ds(base_e + jc * CH, CH)], buf, sem)

            def gather(buf_idx, buf, sem):
                return pltpu.make_async_copy(table.at[buf_idx], buf, sem)

            idx_fetch(0, src0, semi0).start()
            idx_fetch(0, src0, semi0).wait()
            gather(src0, rows0, sem0).start()
            idx_fetch(1, src1, semi1).start()

            def step2(i, carry):
                j0 = 2 * i
                idx_fetch(j0 + 1, src1, semi1).wait()
                gather(src1, rows1, sem1).start()
                gather(src0, rows0, sem0).wait()
                idx_fetch(j0 + 2, src0, semi0).start()
                pltpu.sync_copy(rows0, acc_sh.at[dst_v.at[j0]], add=True)
                idx_fetch(j0 + 2, src0, semi0).wait()
                gather(src0, rows0, sem0).start()
                gather(src1, rows1, sem1).wait()
                idx_fetch(j0 + 3, src1, semi1).start()
                pltpu.sync_copy(rows1, acc_sh.at[dst_v.at[j0 + 1]],
                                add=True)
                return carry

            lax.fori_loop(0, (jn - 1) // 2, step2, 0)
            gather(src0, rows0, sem0).wait()
            idx_fetch(jn - 1, src1, semi1).wait()
            pltpu.sync_copy(rows0, acc_sh.at[dst_v.at[jn - 1]], add=True)

        @pl.when(c == 0)
        def _():
            pltpu.sync_copy(dst_a.at[s], dst_v.at[pl.ds(0, J0)])

        @pl.when(c == 1)
        def _():
            pltpu.sync_copy(dst_b.at[s], dst_v.at[pl.ds(0, J1)])

        plsc.subcore_barrier()

        @pl.when(c == 0)
        def _():
            run_stream(J0, s * (J0 * CH))

        @pl.when(c == 1)
        def _():
            run_stream(J1, NS * J0 * CH + s * (J1 * CH))

        plsc.subcore_barrier()

        NB = SLAB // CH
        pltpu.make_async_copy(acc_sh.at[pl.ds(r0, CH)], rows0, sem0).start()
        for b in range(NB):
            buf = (rows0, rows1)[b % 2]
            nbuf = (rows0, rows1)[(b + 1) % 2]
            sem = (sem0, sem1)[b % 2]
            nsem = (sem0, sem1)[(b + 1) % 2]
            if b + 1 < NB:
                pltpu.make_async_copy(
                    acc_sh.at[pl.ds(r0 + (b + 1) * CH, CH)],
                    nbuf, nsem).start()
            pltpu.make_async_copy(acc_sh.at[pl.ds(r0 + b * CH, CH)],
                                  buf, sem).wait()
            pltpu.sync_copy(buf, psums.at[c].at[pl.ds(r0 + b * CH, CH)])

    return pl.kernel(
        body,
        out_type=(jax.ShapeDtypeStruct((NC, NP, D), jnp.float32),),
        mesh=mesh,
        scratch_types=(
            pltpu.VMEM((J0, CH), jnp.int32),
            pltpu.VMEM((2 * CH,), jnp.int32),
            pltpu.VMEM((2, CH, D), jnp.float32),
            pltpu.SemaphoreType.DMA,
            pltpu.SemaphoreType.DMA,
            pltpu.SemaphoreType.DMA,
            pltpu.SemaphoreType.DMA,
            pltpu.VMEM_SHARED((NP, D), jnp.float32),
        ),
        compiler_params=_sc_params())


@functools.lru_cache(maxsize=None)
def _make_sc_cnt():
    mesh = plsc.VectorSubcoreMesh(core_axis_name="c", subcore_axis_name="s")

    def body(dst3, pcnt, dst_v, hist, cntbuf, slab_v, sem, hist_sh):
        c = lax.axis_index("c")
        s = lax.axis_index("s")
        wid = s * NC + c

        pltpu.sync_copy(dst3.at[wid], dst_v)
        lane = lax.iota(jnp.int32, 16)
        ones16 = jnp.ones((16,), jnp.float32)
        zeros16 = jnp.zeros((16,), jnp.float32)

        for p in range(NP // BINS):
            base = p * BINS

            def zero_row(i, carry):
                hist[carry, pl.ds(i * 16, 16)] = zeros16
                return carry

            for l in range(16):
                lax.fori_loop(0, BINS // 16, zero_row, l)

            def feed(j, carry):
                for k in range(CH // 16):
                    idx = dst_v[j, pl.ds(k * 16, 16)]
                    rel = idx - base
                    m = jnp.logical_and(rel >= 0, rel < BINS)
                    relc = jnp.minimum(jnp.maximum(rel, 0), BINS - 1)
                    plsc.addupdate_scatter(hist, [lane, relc], ones16,
                                           mask=m)
                return carry

            lax.fori_loop(0, J, feed, 0)

            def reduce_cols(ci, carry):
                sl = pl.ds(ci * 16, 16)
                v = hist[0, sl]
                for l in range(1, 16):
                    v = v + hist[l, sl]
                cntbuf[sl] = v
                return carry

            lax.fori_loop(0, BINS // 16, reduce_cols, 0)
            pltpu.sync_copy(cntbuf, hist_sh.at[s].at[0].at[pl.ds(base, BINS)])

        plsc.subcore_barrier()
        r0 = s * SLAB
        pltpu.sync_copy(hist_sh.at[:, 0, pl.ds(r0, SLAB)], slab_v)

        def reduce_slab(ci, carry):
            sl = pl.ds(ci * 16, 16)
            v = slab_v[0, sl]
            for l in range(1, 16):
                v = v + slab_v[l, sl]
            cntbuf[sl] = v
            return carry

        lax.fori_loop(0, SLAB // 16, reduce_slab, 0)
        pltpu.sync_copy(cntbuf.at[pl.ds(0, SLAB)],
                        pcnt.at[c].at[0].at[pl.ds(r0, SLAB)])

    return pl.kernel(
        body,
        out_type=(jax.ShapeDtypeStruct((NC, 1, NP), jnp.float32),),
        mesh=mesh,
        scratch_types=(
            pltpu.VMEM((J, CH), jnp.int32),
            pltpu.VMEM((16, BINS), jnp.float32),
            pltpu.VMEM((BINS,), jnp.float32),
            pltpu.VMEM((16, SLAB), jnp.float32),
            pltpu.SemaphoreType.DMA,
            pltpu.VMEM_SHARED((16, 1, NP), jnp.float32),
        ),
        compiler_params=_sc_params())


def _dense0_body(ps, pc, x, wl, bl, wr, g, b, o, ocnt):
    sums = ps[0, :N, :] + ps[1, :N, :]
    cnt = pc[0, :N, :] + pc[1, :N, :]
    ocnt[:] = cnt
    agg = sums * (1.0 / jnp.maximum(cnt, 1.0))
    h = jax.lax.dot(agg, wl[:], preferred_element_type=jnp.float32)
    h = h + bl[:]
    h = h + jax.lax.dot(x[:], wr[:], preferred_element_type=jnp.float32)
    norm = jnp.sqrt(jnp.sum(h * h, axis=1, keepdims=True))
    h = h / jnp.maximum(norm, 1e-12)
    mu = jnp.mean(h, axis=0, keepdims=True)
    var = jnp.mean((h - mu) * (h - mu), axis=0, keepdims=True)
    h = g[:] * (h - mu) / jnp.sqrt(var + 1e-5) + b[:]
    o[:] = jnp.maximum(h, 0.0)


def _dense1_body(ps, cnt_ref, x, wl, bl, wr, g, b, wfc, bfc, o):
    sums = ps[0, :N, :] + ps[1, :N, :]
    cnt = cnt_ref[:]
    agg = sums * (1.0 / jnp.maximum(cnt, 1.0))
    h = jax.lax.dot(agg, wl[:], preferred_element_type=jnp.float32)
    h = h + bl[:]
    h = h + jax.lax.dot(x[:], wr[:], preferred_element_type=jnp.float32)
    norm = jnp.sqrt(jnp.sum(h * h, axis=1, keepdims=True))
    h = h / jnp.maximum(norm, 1e-12)
    mu = jnp.mean(h, axis=0, keepdims=True)
    var = jnp.mean((h - mu) * (h - mu), axis=0, keepdims=True)
    h = g[:] * (h - mu) / jnp.sqrt(var + 1e-5) + b[:]
    h = jnp.maximum(h, 0.0)
    h = jax.lax.dot(h, wfc[:], preferred_element_type=jnp.float32)
    o[:] = h + bfc[:]


_dense0 = pl.pallas_call(
    _dense0_body,
    out_shape=(jax.ShapeDtypeStruct((N, D), jnp.float32),
               jax.ShapeDtypeStruct((N, 1), jnp.float32)),
)

_dense1 = pl.pallas_call(
    _dense1_body,
    out_shape=jax.ShapeDtypeStruct((N, D), jnp.float32),
)


def kernel(x, edge_index, W_l0, b_l0, W_r0, gamma0, beta0,
           W_l1, b_l1, W_r1, gamma1, beta1, W_fc, b_fc):
    dst = edge_index[0].astype(jnp.int32)
    src = edge_index[1].astype(jnp.int32)
    src1d = jnp.concatenate([src, jnp.zeros((EP - E,), jnp.int32)])
    dstp = jnp.concatenate([dst, jnp.full((EP - E,), N, jnp.int32)])
    dst3 = dstp.reshape(NW, J, CH)
    dst_a = dstp[:NS * J0 * CH].reshape(NS, J0, CH)
    dst_b = dstp[NS * J0 * CH:].reshape(NS, J1, CH)
    z2d = jnp.zeros((SLAB, D), jnp.float32)

    (ps0,) = _make_sc_agg()(x, src1d, dst_a, dst_b, z2d)
    (pc3,) = _make_sc_cnt()(dst3)
    h0, cnt_col = _dense0(ps0, pc3[:, 0, :, None], x, W_l0, b_l0.reshape(1, D),
                          W_r0, gamma0.reshape(1, D), beta0.reshape(1, D))
    (ps1,) = _make_sc_agg()(h0, src1d, dst_a, dst_b, z2d)
    wfc_p = jnp.pad(W_fc, ((0, 0), (0, D - C)))
    bfc_p = jnp.pad(b_fc, (0, D - C)).reshape(1, D)
    out_p = _dense1(ps1, cnt_col, h0, W_l1, b_l1.reshape(1, D), W_r1,
                    gamma1.reshape(1, D), beta1.reshape(1, D),
                    wfc_p, bfc_p)
    return out_p[:, :C]
